# Initial kernel scaffold; baseline (speedup 1.0000x reference)
import jax
import jax.numpy as jnp
from jax.experimental import pallas as pl


def kernel(input_tensor, Ws, bs, token_weights, prelu_a):
    raise NotImplementedError("write your pallas kernel here")


# speedup vs baseline: 2.1997x; 2.1997x over previous
import functools

import jax
import jax.numpy as jnp
from jax import lax
from jax.experimental import pallas as pl
from jax.experimental.pallas import tpu as pltpu

_SELU_ALPHA = 1.6732632423543772
_SELU_SCALE = 1.0507009873554805


def _expm1(x):
    small = jnp.abs(x) < 0.1
    xs = jnp.where(small, x, 0.0)
    series = xs * (1.0 + xs * (0.5 + xs * (1.0 / 6.0 + xs * (1.0 / 24.0 + xs * (1.0 / 120.0)))))
    return jnp.where(small, series, jnp.exp(x) - 1.0)


def _scores_body(x_ref, wt_ref, b_ref, tw_ref, a_ref, comb_ref, sums_ref):
    x = x_ref[...]
    wt = wt_ref[...]
    scores = jnp.dot(x, wt, preferred_element_type=jnp.float32) + b_ref[...]
    a = a_ref[...]
    pre = jnp.where(scores >= 0, scores, a * scores)
    celu = jnp.where(pre > 0, pre, _expm1(pre))
    silu = pre * jax.nn.sigmoid(pre)
    gelu = pre * (lax.erf(pre / jnp.float32(1.4142135623730951)) + 1.0) * 0.5
    ts = celu * silu + gelu
    tw = tw_ref[...]
    wsel = _SELU_SCALE * jnp.where(tw > 0, tw, _SELU_ALPHA * _expm1(tw))
    comb = jnp.dot(ts, wsel.reshape(-1, 1), preferred_element_type=jnp.float32)
    ssum = ts[:, 0]
    for s in range(1, ts.shape[1]):
        ssum = ssum + ts[:, s]
    comb_ref[...] = comb.reshape(1, 1, -1)
    sums_ref[...] = ssum.reshape(1, 1, -1)


def _partner(x, d, lower):
    if d < 128:
        ax, s = 1, d
    else:
        ax, s = 0, d // 128
    sz = x.shape[ax]
    return jnp.where(lower, pltpu.roll(x, sz - s, ax), pltpu.roll(x, s, ax))


def _sort_body(comb_ref, sums_ref, out_ref, *, batch, log2n):
    G, lanes = comb_ref.shape
    rows = G // batch
    n = rows * lanes

    parts = []
    for b in range(batch):
        blk = sums_ref[b * rows:(b + 1) * rows, :]
        m = jnp.max(blk)
        e = jnp.exp(blk - m)
        parts.append(e / jnp.sum(e))
    soft = jnp.concatenate(parts, axis=0)

    c = lax.broadcasted_iota(jnp.int32, (G, lanes), 1)
    g = lax.broadcasted_iota(jnp.int32, (G, lanes), 0)
    i = (g % rows) * lanes + c

    key = comb_ref[...]
    idx = i
    for kk in range(1, log2n + 1):
        asc = (i & (1 << kk)) == 0
        for jj in range(kk - 1, -1, -1):
            d = 1 << jj
            lower = (i & d) == 0
            ky = _partner(key, d, lower)
            iy = _partner(idx, d, lower)
            x_less = (key > ky) | ((key == ky) & (idx < iy))
            take_x = x_less == (asc == lower)
            key = jnp.where(take_x, key, ky)
            idx = jnp.where(take_x, idx, iy)

    soft_u = lax.bitcast_convert_type(soft, jnp.uint32)
    packed = (idx.astype(jnp.uint32) << 16) | (soft_u >> 16)
    for kk in range(1, log2n + 1):
        asc = (i & (1 << kk)) == 0
        for jj in range(kk - 1, -1, -1):
            d = 1 << jj
            lower = (i & d) == 0
            py = _partner(packed, d, lower)
            x_less = packed < py
            take_x = x_less == (asc == lower)
            packed = jnp.where(take_x, packed, py)

    val = lax.bitcast_convert_type(packed << 16, jnp.float32)
    out_ref[...] = 1.0 + val


def kernel(input_tensor, Ws, bs, token_weights, prelu_a):
    B, L, D = input_tensor.shape
    S = Ws.shape[0]
    LC = 2048
    BL = B * L
    x2 = input_tensor.reshape(BL, D)
    wt = Ws.T
    b2 = bs.reshape(1, S)
    tw2 = token_weights.reshape(1, S)
    a2 = prelu_a.reshape(1, 1)

    nblk = BL // LC
    comb, sums = pl.pallas_call(
        _scores_body,
        grid=(nblk,),
        in_specs=[
            pl.BlockSpec((LC, D), lambda i: (i, 0)),
            pl.BlockSpec((D, S), lambda i: (0, 0)),
            pl.BlockSpec((1, S), lambda i: (0, 0)),
            pl.BlockSpec((1, S), lambda i: (0, 0)),
            pl.BlockSpec((1, 1), lambda i: (0, 0)),
        ],
        out_specs=[
            pl.BlockSpec((1, 1, LC), lambda i: (i, 0, 0)),
            pl.BlockSpec((1, 1, LC), lambda i: (i, 0, 0)),
        ],
        out_shape=[jax.ShapeDtypeStruct((nblk, 1, LC), jnp.float32)] * 2,
    )(x2, wt, b2, tw2, a2)

    G = BL // 128
    log2n = (L - 1).bit_length()
    out = pl.pallas_call(
        functools.partial(_sort_body, batch=B, log2n=log2n),
        out_shape=jax.ShapeDtypeStruct((G, 128), jnp.float32),
    )(comb.reshape(G, 128), sums.reshape(G, 128))

    return out.reshape(B, L, 1)
